# baseline (speedup 1.0000x reference)
import functools

import jax
import jax.numpy as jnp
from jax import lax
from jax.experimental import pallas as pl
from jax.experimental.pallas import tpu as pltpu
from jax.experimental.pallas import tpu_sc as plsc

NUM_EMBEDDINGS = 1000000
EMBEDDING_DIM = 32
BATCH = 16384
FIELDS = 26

NC, NS = 2, 16
NW = NC * NS
B_TOT = BATCH * FIELDS
BPW = B_TOT // NW
UPW = BPW // 128

NBLK = (NUM_EMBEDDINGS + 127) // 128
PK_ROWS = ((NBLK * 128) // 4) + 12
BLK_BASE = NBLK // NW
BLK_REM = NBLK - BLK_BASE * NW

_MESH = plsc.VectorSubcoreMesh(core_axis_name="c", subcore_axis_name="s")
_PARAMS = pltpu.CompilerParams(
    use_tc_tiling_on_sc=True, needs_layout_passes=False
)


def _wid():
    return lax.axis_index("s") * NC + lax.axis_index("c")


def _transpose_block(src, dst, iota, n_pr):
    for pr in range(n_pr):
        for cg in range(8):
            rows = iota + (cg % 2) * 16
            cols = jnp.full((16,), 4 * pr + cg // 2, jnp.int32)
            v = plsc.load_gather(src, [rows, cols])
            dst[pr, pl.ds(cg * 16, 16)] = v


@functools.partial(
    pl.kernel,
    out_type=jax.ShapeDtypeStruct((PK_ROWS, 128), jnp.float32),
    mesh=_MESH,
    compiler_params=_PARAMS,
    scratch_types=[
        pltpu.VMEM((32, 131), jnp.float32),
        pltpu.VMEM((32, 131), jnp.float32),
        pltpu.VMEM((32, 128), jnp.float32),
        pltpu.VMEM((32, 128), jnp.float32),
        pltpu.SemaphoreType.DMA,
        pltpu.SemaphoreType.DMA,
        pltpu.SemaphoreType.DMA,
        pltpu.SemaphoreType.DMA,
    ],
)
def _retile(tt_hbm, pk_hbm, in0, in1, ot0, ot1, gi0, gi1, so0, so1):
    w = _wid()
    start = w * BLK_BASE + jnp.minimum(w, BLK_REM)
    nb = BLK_BASE + jnp.where(w < BLK_REM, 1, 0)
    iota = lax.iota(jnp.int32, 16)
    ins = (in0, in1)
    ots = (ot0, ot1)
    gis = (gi0, gi1)
    sos = (so0, so1)

    def fetch(b, k):
        return pltpu.async_copy(
            tt_hbm.at[:, pl.ds(b * 128, 128)], ins[k].at[:, pl.ds(0, 128)],
            gis[k],
        )

    def put(b, k):
        return pltpu.async_copy(
            ots[k], pk_hbm.at[pl.ds(b * 32, 32), :], sos[k]
        )

    def wait_fetch(k):
        pltpu.make_async_copy(
            tt_hbm.at[:, pl.ds(0, 128)], ins[k].at[:, pl.ds(0, 128)], gis[k]
        ).wait()

    def wait_put(k):
        pltpu.make_async_copy(ots[k], pk_hbm.at[pl.ds(0, 32), :], sos[k]).wait()

    fetch(start, 0)

    @pl.when(1 < nb)
    def _():
        fetch(start + 1, 1)

    def body(t, carry):
        b0 = start + 2 * t
        b1 = b0 + 1

        wait_fetch(0)
        _transpose_block(ins[0], ots[0], iota, 32)
        put(b0, 0)

        @pl.when(b0 + 2 < start + nb)
        def _():
            wait_put(0)
            fetch(b0 + 2, 0)

        @pl.when(b1 < start + nb)
        def _():
            wait_fetch(1)
            _transpose_block(ins[1], ots[1], iota, 32)
            put(b1, 1)

            @pl.when(b1 + 2 < start + nb)
            def _():
                wait_put(1)
                fetch(b1 + 2, 1)

        return carry

    lax.fori_loop(0, (nb + 1) // 2, body, 0)
    @pl.when(nb >= 1)
    def _():
        wait_put(0)

    @pl.when(nb >= 2)
    def _():
        wait_put(1)


@functools.partial(
    pl.kernel,
    out_type=jax.ShapeDtypeStruct((FIELDS, EMBEDDING_DIM, BATCH), jnp.float32),
    mesh=_MESH,
    compiler_params=_PARAMS,
    scratch_types=[
        pltpu.VMEM((BPW,), jnp.int32),
        pltpu.VMEM((BPW,), jnp.int32),
        pltpu.VMEM((128, 129), jnp.float32),
        pltpu.VMEM((128, 129), jnp.float32),
        pltpu.VMEM((32, 129), jnp.float32),
        pltpu.SemaphoreType.DMA,
        pltpu.SemaphoreType.DMA,
    ],
)
def _lookup(xf_hbm, pk_hbm, out_hbm, pidx_v, off_v, buf0, buf1, ot, g0, g1):
    w = _wid()
    base_u = w * UPW
    iota = lax.iota(jnp.int32, 16)
    bufs = (buf0, buf1)
    sems = (g0, g1)

    pltpu.sync_copy(xf_hbm.at[pl.ds(w * BPW, BPW)], pidx_v)

    def prep(s, carry):
        v = pidx_v[pl.ds(s * 16, 16)]
        off_v[pl.ds(s * 16, 16)] = (v & 3) * 32
        pidx_v[pl.ds(s * 16, 16)] = v >> 2
        return carry

    lax.fori_loop(0, BPW // 16, prep, 0)

    def fire(u_loc, k):
        return pltpu.async_copy(
            pk_hbm.at[pidx_v.at[pl.ds(u_loc * 128, 128)]],
            bufs[k].at[:, pl.ds(0, 128)],
            sems[k],
        )

    def wait_g(k):
        pltpu.make_async_copy(
            pk_hbm.at[pidx_v.at[pl.ds(0, 128)]],
            bufs[k].at[:, pl.ds(0, 128)],
            sems[k],
        ).wait()

    def extract_store(u_loc, k):
        u = base_u + u_loc
        f = u >> 7
        blk = u & 127
        buf = bufs[k]
        for g in range(8):
            offs = off_v[pl.ds(u_loc * 128 + g * 16, 16)]
            rows = iota + g * 16
            for j in range(EMBEDDING_DIM):
                v = plsc.load_gather(buf, [rows, offs + j])
                ot[j, pl.ds(g * 16, 16)] = v
        pltpu.sync_copy(
            ot.at[:, pl.ds(0, 128)], out_hbm.at[f, :, pl.ds(blk * 128, 128)]
        )

    fire(0, 0)

    def body(t, carry):
        u0 = 2 * t
        u1 = u0 + 1
        fire(u1, 1)
        wait_g(0)
        extract_store(u0, 0)

        @pl.when(t < UPW // 2 - 1)
        def _():
            fire(u0 + 2, 0)

        wait_g(1)
        extract_store(u1, 1)
        return carry

    lax.fori_loop(0, UPW // 2, body, 0)


def kernel(x, table):
    xf = x.T.reshape(-1)
    pk = _retile(table.T)
    out3 = _lookup(xf, pk)
    return out3.transpose(2, 0, 1)


# speedup vs baseline: 1.6733x; 1.6733x over previous
import functools

import jax
import jax.numpy as jnp
from jax import lax
from jax.experimental import pallas as pl
from jax.experimental.pallas import tpu as pltpu
from jax.experimental.pallas import tpu_sc as plsc

NUM_EMBEDDINGS = 1000000
EMBEDDING_DIM = 32
BATCH = 16384
FIELDS = 26

NC, NS = 2, 16
NW = NC * NS
B_TOT = BATCH * FIELDS
BPW = B_TOT // NW
UPW = BPW // 128

NBLK = (NUM_EMBEDDINGS + 127) // 128
PK_ROWS = ((NBLK * 128) // 4) + 12
BLK_BASE = NBLK // NW
BLK_REM = NBLK - BLK_BASE * NW

_MESH = plsc.VectorSubcoreMesh(core_axis_name="c", subcore_axis_name="s")
_PARAMS = pltpu.CompilerParams(
    use_tc_tiling_on_sc=True, needs_layout_passes=False
)


def _wid():
    return lax.axis_index("s") * NC + lax.axis_index("c")


def _transpose_block(src, dst, iota, n_pr):
    for pr in range(n_pr):
        for cg in range(8):
            rows = (iota >> 2) + 4 * cg
            cols = (iota & 3) + 4 * pr
            v = plsc.load_gather(src, [rows, cols])
            dst[pr, pl.ds(cg * 16, 16)] = v


@functools.partial(
    pl.kernel,
    out_type=jax.ShapeDtypeStruct((PK_ROWS, 128), jnp.float32),
    mesh=_MESH,
    compiler_params=_PARAMS,
    scratch_types=[
        pltpu.VMEM((32, 131), jnp.float32),
        pltpu.VMEM((32, 131), jnp.float32),
        pltpu.VMEM((32, 128), jnp.float32),
        pltpu.VMEM((32, 128), jnp.float32),
        pltpu.SemaphoreType.DMA,
        pltpu.SemaphoreType.DMA,
        pltpu.SemaphoreType.DMA,
        pltpu.SemaphoreType.DMA,
    ],
)
def _retile(tt_hbm, pk_hbm, in0, in1, ot0, ot1, gi0, gi1, so0, so1):
    w = _wid()
    start = w * BLK_BASE + jnp.minimum(w, BLK_REM)
    nb = BLK_BASE + jnp.where(w < BLK_REM, 1, 0)
    iota = lax.iota(jnp.int32, 16)
    ins = (in0, in1)
    ots = (ot0, ot1)
    gis = (gi0, gi1)
    sos = (so0, so1)

    def fetch(b, k):
        return pltpu.async_copy(
            tt_hbm.at[:, pl.ds(b * 128, 128)], ins[k].at[:, pl.ds(0, 128)],
            gis[k],
        )

    def put(b, k):
        return pltpu.async_copy(
            ots[k], pk_hbm.at[pl.ds(b * 32, 32), :], sos[k]
        )

    def wait_fetch(k):
        pltpu.make_async_copy(
            tt_hbm.at[:, pl.ds(0, 128)], ins[k].at[:, pl.ds(0, 128)], gis[k]
        ).wait()

    def wait_put(k):
        pltpu.make_async_copy(ots[k], pk_hbm.at[pl.ds(0, 32), :], sos[k]).wait()

    fetch(start, 0)

    @pl.when(1 < nb)
    def _():
        fetch(start + 1, 1)

    def body(t, carry):
        b0 = start + 2 * t
        b1 = b0 + 1

        wait_fetch(0)
        _transpose_block(ins[0], ots[0], iota, 32)
        put(b0, 0)

        @pl.when(b0 + 2 < start + nb)
        def _():
            wait_put(0)
            fetch(b0 + 2, 0)

        @pl.when(b1 < start + nb)
        def _():
            wait_fetch(1)
            _transpose_block(ins[1], ots[1], iota, 32)
            put(b1, 1)

            @pl.when(b1 + 2 < start + nb)
            def _():
                wait_put(1)
                fetch(b1 + 2, 1)

        return carry

    lax.fori_loop(0, (nb + 1) // 2, body, 0)
    @pl.when(nb >= 1)
    def _():
        wait_put(0)

    @pl.when(nb >= 2)
    def _():
        wait_put(1)


@functools.partial(
    pl.kernel,
    out_type=jax.ShapeDtypeStruct((FIELDS, EMBEDDING_DIM, BATCH), jnp.float32),
    mesh=_MESH,
    compiler_params=_PARAMS,
    scratch_types=[
        pltpu.VMEM((BPW,), jnp.int32),
        pltpu.VMEM((BPW,), jnp.int32),
        pltpu.VMEM((128, 128), jnp.float32),
        pltpu.VMEM((128, 128), jnp.float32),
        pltpu.VMEM((32, 129), jnp.float32),
        pltpu.SemaphoreType.DMA,
        pltpu.SemaphoreType.DMA,
    ],
)
def _lookup(xf_hbm, pk_hbm, out_hbm, pidx_v, off_v, buf0, buf1, ot, g0, g1):
    w = _wid()
    base_u = w * UPW
    iota = lax.iota(jnp.int32, 16)
    bufs = (buf0, buf1)
    sems = (g0, g1)

    pltpu.sync_copy(xf_hbm.at[pl.ds(w * BPW, BPW)], pidx_v)

    def prep(s, carry):
        v = pidx_v[pl.ds(s * 16, 16)]
        off_v[pl.ds(s * 16, 16)] = v & 3
        pidx_v[pl.ds(s * 16, 16)] = v >> 2
        return carry

    lax.fori_loop(0, BPW // 16, prep, 0)

    def fire(u_loc, k):
        return pltpu.async_copy(
            pk_hbm.at[pidx_v.at[pl.ds(u_loc * 128, 128)]], bufs[k], sems[k]
        )

    def wait_g(k):
        pltpu.make_async_copy(
            pk_hbm.at[pidx_v.at[pl.ds(0, 128)]], bufs[k], sems[k]
        ).wait()

    def extract_store(u_loc, k):
        u = base_u + u_loc
        f = u >> 7
        blk = u & 127
        buf = bufs[k]
        for g in range(8):
            qs = off_v[pl.ds(u_loc * 128 + g * 16, 16)]
            rows = iota + g * 16
            for j in range(EMBEDDING_DIM):
                v = plsc.load_gather(buf, [rows, qs + 4 * j])
                ot[j, pl.ds(g * 16, 16)] = v
        pltpu.sync_copy(
            ot.at[:, pl.ds(0, 128)], out_hbm.at[f, :, pl.ds(blk * 128, 128)]
        )

    fire(0, 0)

    def body(t, carry):
        u0 = 2 * t
        u1 = u0 + 1
        fire(u1, 1)
        wait_g(0)
        extract_store(u0, 0)

        @pl.when(t < UPW // 2 - 1)
        def _():
            fire(u0 + 2, 0)

        wait_g(1)
        extract_store(u1, 1)
        return carry

    lax.fori_loop(0, UPW // 2, body, 0)


def kernel(x, table):
    xf = x.T.reshape(-1)
    pk = _retile(table.T)
    out3 = _lookup(xf, pk)
    return out3.transpose(2, 0, 1)


# speedup vs baseline: 1.6741x; 1.0005x over previous
import functools

import jax
import jax.numpy as jnp
from jax import lax
from jax.experimental import pallas as pl
from jax.experimental.pallas import tpu as pltpu
from jax.experimental.pallas import tpu_sc as plsc

NUM_EMBEDDINGS = 1000000
EMBEDDING_DIM = 32
BATCH = 16384
FIELDS = 26

NC, NS = 2, 16
NW = NC * NS
B_TOT = BATCH * FIELDS
BPW = B_TOT // NW
UPW = BPW // 128

NBLK = (NUM_EMBEDDINGS + 127) // 128
PK_ROWS = ((NBLK * 128) // 4) + 12
BLK_BASE = NBLK // NW
BLK_REM = NBLK - BLK_BASE * NW

_MESH = plsc.VectorSubcoreMesh(core_axis_name="c", subcore_axis_name="s")
_PARAMS = pltpu.CompilerParams(
    use_tc_tiling_on_sc=True, needs_layout_passes=False
)


def _wid():
    return lax.axis_index("s") * NC + lax.axis_index("c")


def _transpose_block(src, dst, iota, n_pr):
    for pr in range(n_pr):
        for cg in range(8):
            rows = (iota >> 2) + 4 * cg
            cols = (iota & 3) + 4 * pr
            v = plsc.load_gather(src, [rows, cols])
            dst[pr, pl.ds(cg * 16, 16)] = v


@functools.partial(
    pl.kernel,
    out_type=jax.ShapeDtypeStruct((PK_ROWS, 128), jnp.float32),
    mesh=_MESH,
    compiler_params=_PARAMS,
    scratch_types=[
        pltpu.VMEM((32, 132), jnp.float32),
        pltpu.VMEM((32, 132), jnp.float32),
        pltpu.VMEM((32, 128), jnp.float32),
        pltpu.VMEM((32, 128), jnp.float32),
        pltpu.SemaphoreType.DMA,
        pltpu.SemaphoreType.DMA,
        pltpu.SemaphoreType.DMA,
        pltpu.SemaphoreType.DMA,
    ],
)
def _retile(tt_hbm, pk_hbm, in0, in1, ot0, ot1, gi0, gi1, so0, so1):
    w = _wid()
    start = w * BLK_BASE + jnp.minimum(w, BLK_REM)
    nb = BLK_BASE + jnp.where(w < BLK_REM, 1, 0)
    iota = lax.iota(jnp.int32, 16)
    ins = (in0, in1)
    ots = (ot0, ot1)
    gis = (gi0, gi1)
    sos = (so0, so1)

    def fetch(b, k):
        return pltpu.async_copy(
            tt_hbm.at[:, pl.ds(b * 128, 128)], ins[k].at[:, pl.ds(0, 128)],
            gis[k],
        )

    def put(b, k):
        return pltpu.async_copy(
            ots[k], pk_hbm.at[pl.ds(b * 32, 32), :], sos[k]
        )

    def wait_fetch(k):
        pltpu.make_async_copy(
            tt_hbm.at[:, pl.ds(0, 128)], ins[k].at[:, pl.ds(0, 128)], gis[k]
        ).wait()

    def wait_put(k):
        pltpu.make_async_copy(ots[k], pk_hbm.at[pl.ds(0, 32), :], sos[k]).wait()

    fetch(start, 0)

    @pl.when(1 < nb)
    def _():
        fetch(start + 1, 1)

    def body(t, carry):
        b0 = start + 2 * t
        b1 = b0 + 1

        wait_fetch(0)
        _transpose_block(ins[0], ots[0], iota, 32)
        put(b0, 0)

        @pl.when(b0 + 2 < start + nb)
        def _():
            wait_put(0)
            fetch(b0 + 2, 0)

        @pl.when(b1 < start + nb)
        def _():
            wait_fetch(1)
            _transpose_block(ins[1], ots[1], iota, 32)
            put(b1, 1)

            @pl.when(b1 + 2 < start + nb)
            def _():
                wait_put(1)
                fetch(b1 + 2, 1)

        return carry

    lax.fori_loop(0, (nb + 1) // 2, body, 0)
    @pl.when(nb >= 1)
    def _():
        wait_put(0)

    @pl.when(nb >= 2)
    def _():
        wait_put(1)


@functools.partial(
    pl.kernel,
    out_type=jax.ShapeDtypeStruct((FIELDS, EMBEDDING_DIM, BATCH), jnp.float32),
    mesh=_MESH,
    compiler_params=_PARAMS,
    scratch_types=[
        pltpu.VMEM((BPW,), jnp.int32),
        pltpu.VMEM((BPW,), jnp.int32),
        pltpu.VMEM((128, 128), jnp.float32),
        pltpu.VMEM((128, 128), jnp.float32),
        pltpu.VMEM((32, 129), jnp.float32),
        pltpu.SemaphoreType.DMA,
        pltpu.SemaphoreType.DMA,
    ],
)
def _lookup(xf_hbm, pk_hbm, out_hbm, pidx_v, off_v, buf0, buf1, ot, g0, g1):
    w = _wid()
    base_u = w * UPW
    iota = lax.iota(jnp.int32, 16)
    bufs = (buf0, buf1)
    sems = (g0, g1)

    pltpu.sync_copy(xf_hbm.at[pl.ds(w * BPW, BPW)], pidx_v)

    def prep(s, carry):
        v = pidx_v[pl.ds(s * 16, 16)]
        off_v[pl.ds(s * 16, 16)] = v & 3
        pidx_v[pl.ds(s * 16, 16)] = v >> 2
        return carry

    lax.fori_loop(0, BPW // 16, prep, 0)

    def fire(u_loc, k):
        return pltpu.async_copy(
            pk_hbm.at[pidx_v.at[pl.ds(u_loc * 128, 128)]], bufs[k], sems[k]
        )

    def wait_g(k):
        pltpu.make_async_copy(
            pk_hbm.at[pidx_v.at[pl.ds(0, 128)]], bufs[k], sems[k]
        ).wait()

    def extract_store(u_loc, k):
        u = base_u + u_loc
        f = u >> 7
        blk = u & 127
        buf = bufs[k]
        for g in range(8):
            qs = off_v[pl.ds(u_loc * 128 + g * 16, 16)]
            rows = iota + g * 16
            for j in range(EMBEDDING_DIM):
                v = plsc.load_gather(buf, [rows, qs + 4 * j])
                ot[j, pl.ds(g * 16, 16)] = v
        pltpu.sync_copy(
            ot.at[:, pl.ds(0, 128)], out_hbm.at[f, :, pl.ds(blk * 128, 128)]
        )

    fire(0, 0)

    def body(t, carry):
        u0 = 2 * t
        u1 = u0 + 1
        fire(u1, 1)
        wait_g(0)
        extract_store(u0, 0)

        @pl.when(t < UPW // 2 - 1)
        def _():
            fire(u0 + 2, 0)

        wait_g(1)
        extract_store(u1, 1)
        return carry

    lax.fori_loop(0, UPW // 2, body, 0)


def kernel(x, table):
    xf = x.T.reshape(-1)
    pk = _retile(table.T)
    out3 = _lookup(xf, pk)
    return out3.transpose(2, 0, 1)


# speedup vs baseline: 1.8098x; 1.0811x over previous
import functools

import jax
import jax.numpy as jnp
from jax import lax
from jax.experimental import pallas as pl
from jax.experimental.pallas import tpu as pltpu
from jax.experimental.pallas import tpu_sc as plsc

NUM_EMBEDDINGS = 1000000
EMBEDDING_DIM = 32
BATCH = 16384
FIELDS = 26

NC, NS = 2, 16
NW = NC * NS
B_TOT = BATCH * FIELDS
BPW = B_TOT // NW
CH = 1664
NCH = BPW // CH
assert CH * NCH == BPW


@functools.partial(
    pl.kernel,
    out_type=jax.ShapeDtypeStruct((B_TOT, EMBEDDING_DIM), jnp.float32),
    mesh=plsc.VectorSubcoreMesh(core_axis_name="c", subcore_axis_name="s"),
    compiler_params=pltpu.CompilerParams(use_tc_tiling_on_sc=False),
    scratch_types=[
        pltpu.VMEM((BPW,), jnp.int32),
        pltpu.VMEM((CH, EMBEDDING_DIM), jnp.float32),
        pltpu.VMEM((CH, EMBEDDING_DIM), jnp.float32),
        pltpu.SemaphoreType.DMA,
        pltpu.SemaphoreType.DMA,
        pltpu.SemaphoreType.DMA,
        pltpu.SemaphoreType.DMA,
    ],
)
def _emb_lookup(x_hbm, table_hbm, out_hbm, idx_v, buf0, buf1, g0, g1, s0, s1):
    wid = lax.axis_index("s") * NC + lax.axis_index("c")
    base = wid * BPW
    pltpu.sync_copy(x_hbm.at[pl.ds(base, BPW)], idx_v)

    bufs = (buf0, buf1)
    gsems = (g0, g1)
    ssems = (s0, s1)

    def gather(ch):
        b = ch % 2
        return pltpu.async_copy(
            table_hbm.at[idx_v.at[pl.ds(ch * CH, CH)]], bufs[b], gsems[b]
        )

    def store(ch):
        b = ch % 2
        return pltpu.async_copy(
            bufs[b], out_hbm.at[pl.ds(base + ch * CH, CH)], ssems[b]
        )

    g = [None] * NCH
    s = [None] * NCH
    g[0] = gather(0)
    g[1] = gather(1)
    for ch in range(NCH):
        g[ch].wait()
        s[ch] = store(ch)
        if ch + 2 < NCH:
            s[ch].wait()
            g[ch + 2] = gather(ch + 2)
    s[NCH - 2].wait()
    s[NCH - 1].wait()


def kernel(x, table):
    flat = _emb_lookup(x.reshape(-1), table)
    return flat.reshape(BATCH, FIELDS, EMBEDDING_DIM)
